# baseline (speedup 1.0000x reference)
import jax
import jax.numpy as jnp
from jax import lax
from jax.experimental import pallas as pl
from jax.experimental.pallas import tpu as pltpu
from jax.experimental.pallas import tpu_sc as plsc

BATCH = 16384
N_FIELDS = 26
OUT_DIM = 32
NC = 2
NS = 16
NW = NC * NS
BPW = BATCH // NW
PER_W = BPW * N_FIELDS
CHUNK = 128
SUBC = BPW // CHUNK
L = 16


def _body(table_hbm, idx_hbm, out_hbm,
          idx_v, pid_v, gb0, gb1, ob0, ob1,
          gs0, gs1, ws0, ws1, isem):
    wid = lax.axis_index("s") * NC + lax.axis_index("c")
    b0 = wid * BPW

    for f in range(N_FIELDS):
        pltpu.async_copy(
            idx_hbm.at[pl.ds(f * BATCH + b0, BPW)],
            idx_v.at[pl.ds(f * BPW, BPW)], isem)
    pltpu.make_async_copy(idx_hbm.at[pl.ds(0, PER_W)], idx_v, isem).wait()

    def shift(i, c):
        pid_v[pl.ds(i * L, L)] = lax.shift_right_logical(idx_v[pl.ds(i * L, L)], 2)
        return c
    lax.fori_loop(0, PER_W // L, shift, 0)

    gbufs = (gb0, gb1)
    gsems = (gs0, gs1)
    obufs = (ob0, ob1)
    wsems = (ws0, ws1)

    def fire(t, p):
        pltpu.async_copy(
            table_hbm.at[pid_v.at[pl.ds(t * CHUNK, CHUNK)]],
            gbufs[p], gsems[p])

    def wait_gather(p):
        pltpu.make_async_copy(table_hbm.at[pl.ds(0, CHUNK)],
                              gbufs[p], gsems[p]).wait()

    lanes = lax.iota(jnp.int32, L)
    def extract(t, gp, op):
        gb = gbufs[gp]
        ob = obufs[op]

        def gbody(g, carry):
            jv = idx_v[pl.ds(t * CHUNK + g * L, L)]
            qv = (jv & 3) * OUT_DIM
            rows = lanes + g * L
            for h in range(2):
                qh = qv + (h * L)
                for k in range(L):
                    rot = (lanes + k) & (L - 1)
                    vals = plsc.load_gather(gb, [rows, qh + rot])
                    plsc.store_scatter(ob, [rot + (h * L), rows], vals)
            return carry

        lax.fori_loop(0, CHUNK // L, gbody, 0)

    def drain_wb(p):
        pltpu.make_async_copy(
            obufs[p], out_hbm.at[0, :, pl.ds(0, CHUNK)], wsems[p]).wait()

    fire(0, 0)
    fire(1, 1)
    n_chunks = N_FIELDS * SUBC

    def fpair(j, carry):
        for f_par in (0, 1):
            f = 2 * j + f_par
            for s in range(SUBC):
                t = f * SUBC + s
                gp = s % 2
                op = s % 2
                wait_gather(gp)
                @pl.when(t >= 2)
                def _():
                    drain_wb(op)
                extract(t, gp, op)
                @pl.when(t + 2 < n_chunks)
                def _():
                    fire(t + 2, s % 2)
                pltpu.async_copy(
                    obufs[op],
                    out_hbm.at[f, :, pl.ds(b0 + s * CHUNK, CHUNK)],
                    wsems[op])
        return carry

    lax.fori_loop(0, N_FIELDS // 2, fpair, 0)

    drain_wb(0)
    drain_wb(1)


def kernel(x, table):
    idx = x.T.reshape(-1).astype(jnp.int32)
    table2 = table.reshape(250000, 128)
    mesh = plsc.VectorSubcoreMesh(core_axis_name="c", subcore_axis_name="s")
    f = pl.kernel(
        _body,
        mesh=mesh,
        out_type=jax.ShapeDtypeStruct((N_FIELDS, OUT_DIM, BATCH), jnp.float32),
        scratch_types=[
            pltpu.VMEM((PER_W,), jnp.int32),
            pltpu.VMEM((PER_W,), jnp.int32),
            pltpu.VMEM((CHUNK, 128), jnp.float32),
            pltpu.VMEM((CHUNK, 128), jnp.float32),
            pltpu.VMEM((OUT_DIM, CHUNK), jnp.float32),
            pltpu.VMEM((OUT_DIM, CHUNK), jnp.float32),
            pltpu.SemaphoreType.DMA,
            pltpu.SemaphoreType.DMA,
            pltpu.SemaphoreType.DMA,
            pltpu.SemaphoreType.DMA,
            pltpu.SemaphoreType.DMA,
        ],
        compiler_params=pltpu.CompilerParams(
            use_tc_tiling_on_sc=True, needs_layout_passes=False),
    )
    out = f(table2, idx)
    return out.transpose(2, 0, 1)


# speedup vs baseline: 1.5189x; 1.5189x over previous
import jax
import jax.numpy as jnp
from jax import lax
from jax.experimental import pallas as pl
from jax.experimental.pallas import tpu as pltpu
from jax.experimental.pallas import tpu_sc as plsc

BATCH = 16384
N_FIELDS = 26
OUT_DIM = 32
NC = 2
NS = 16
NW = NC * NS
BPW = BATCH // NW
PER_W = BPW * N_FIELDS
CHUNK = 128
SUBC = BPW // CHUNK
L = 16


def _body(table_hbm, idx_hbm, out_hbm,
          idx_v, pid_v, gb0, gb1, ob0, ob1,
          gs0, gs1, ws0, ws1, isem):
    wid = lax.axis_index("s") * NC + lax.axis_index("c")
    b0 = wid * BPW

    for f in range(N_FIELDS):
        pltpu.async_copy(
            idx_hbm.at[pl.ds(f * BATCH + b0, BPW)],
            idx_v.at[pl.ds(f * BPW, BPW)], isem)
    pltpu.make_async_copy(idx_hbm.at[pl.ds(0, PER_W)], idx_v, isem).wait()

    def shift(i, c):
        pid_v[pl.ds(i * L, L)] = lax.shift_right_logical(idx_v[pl.ds(i * L, L)], 2)
        return c
    lax.fori_loop(0, PER_W // L, shift, 0)

    gbufs = (gb0, gb1)
    gsems = (gs0, gs1)
    obufs = (ob0, ob1)
    wsems = (ws0, ws1)

    def fire(t, p):
        pltpu.async_copy(
            table_hbm.at[pid_v.at[pl.ds(t * CHUNK, CHUNK)]],
            gbufs[p], gsems[p])

    def wait_gather(p):
        pltpu.make_async_copy(table_hbm.at[pl.ds(0, CHUNK)],
                              gbufs[p], gsems[p]).wait()

    lanes = lax.iota(jnp.int32, L)
    def extract(t, gp, op):
        gb = gbufs[gp]
        ob = obufs[op]

        def gbody(g, carry):
            jv = idx_v[pl.ds(t * CHUNK + g * L, L)]
            qv = (jv & 3) * OUT_DIM
            rows = lanes + g * L
            for h in range(2):
                qh = qv + (h * L)
                for k in range(L):
                    rot = (lanes + k) & (L - 1)
                    vals = plsc.load_gather(gb, [rows, qh + rot])
                    plsc.store_scatter(ob, [rot + (h * L), rows], vals)
            return carry

        lax.fori_loop(0, CHUNK // L, gbody, 0)

    def drain_wb(p):
        pltpu.make_async_copy(
            obufs[p], out_hbm.at[0, :, pl.ds(0, CHUNK)], wsems[p]).wait()

    fire(0, 0)
    fire(1, 1)
    n_chunks = N_FIELDS * SUBC

    def fpair(j, carry):
        for f_par in (0, 1):
            f = 2 * j + f_par
            for s in range(SUBC):
                t = f * SUBC + s
                gp = s % 2
                op = s % 2
                wait_gather(gp)
                @pl.when(t >= 2)
                def _():
                    drain_wb(op)
                extract(t, gp, op)
                @pl.when(t + 2 < n_chunks)
                def _():
                    fire(t + 2, s % 2)
                pltpu.async_copy(
                    obufs[op],
                    out_hbm.at[f, :, pl.ds(b0 + s * CHUNK, CHUNK)],
                    wsems[op])
        return carry

    lax.fori_loop(0, N_FIELDS // 2, fpair, 0)

    drain_wb(0)
    drain_wb(1)


NBLK = 7812
BLK_PER_W = 246


def _transpose_body(tt_hbm, tail_hbm, flat_hbm, ab0, ab1, tb0, tb1,
                    is0, is1, ws0, ws1):
    wid = lax.axis_index("s") * NC + lax.axis_index("c")
    base = wid * BLK_PER_W

    abufs = (ab0, ab1)
    tbufs = (tb0, tb1)
    isems = (is0, is1)
    wsems = (ws0, ws1)

    lanes = lax.iota(jnp.int32, L)
    l4 = lax.shift_right_logical(lanes, 2)
    la3 = (lanes & 3) * OUT_DIM
    cvecs = [lanes + c * L for c in range(8)]
    rvecs = [l4 + (c * 4) for c in range(8)]

    def fire_in(bi, p):
        pltpu.async_copy(tt_hbm.at[:, pl.ds(bi * CHUNK, CHUNK)],
                         abufs[p], isems[p])

    def wait_in(p):
        pltpu.make_async_copy(tt_hbm.at[:, pl.ds(0, CHUNK)],
                              abufs[p], isems[p]).wait()

    def drain_wb(p):
        pltpu.make_async_copy(tbufs[p], flat_hbm.at[pl.ds(0, OUT_DIM), :],
                              wsems[p]).wait()

    def shuffle(p):
        ab, tb = abufs[p], tbufs[p]

        def kstep(k, carry):
            rot = (lanes + k) & (L - 1)
            for h in range(2):
                hrot = rot + h * L
                colv = la3 + (h * L) + rot
                for c in range(8):
                    vals = plsc.load_gather(ab, [hrot, cvecs[c]])
                    plsc.store_scatter(tb, [rvecs[c], colv], vals)
            return carry

        lax.fori_loop(0, L, kstep, 0)

    @pl.when(wid == NW - 1)
    def _():
        pltpu.sync_copy(tail_hbm, ab0)

        def tstep(k, carry):
            rot = (lanes + k) & (L - 1)
            for h in range(2):
                hrot = rot + h * L
                colv = la3 + (h * L) + rot
                for c in range(4):
                    vals = plsc.load_gather(ab0, [hrot, cvecs[c]])
                    plsc.store_scatter(tb0, [rvecs[c], colv], vals)
            return carry

        lax.fori_loop(0, L, tstep, 0)
        pltpu.sync_copy(tb0.at[pl.ds(0, 16), :],
                        flat_hbm.at[pl.ds(NBLK * 32, 16), :])

    @pl.when(base < NBLK)
    def _():
        fire_in(base, 0)
    @pl.when(base + 1 < NBLK)
    def _():
        fire_in(base + 1, 1)

    def pair(i, carry):
        for p in (0, 1):
            j = 2 * i + p
            bi = base + j
            @pl.when(bi < NBLK)
            def _():
                wait_in(p)
                @pl.when(j >= 2)
                def _():
                    drain_wb(p)
                shuffle(p)
                @pl.when((j + 2 < BLK_PER_W) & (bi + 2 < NBLK))
                def _():
                    fire_in(bi + 2, p)
                pltpu.async_copy(tbufs[p],
                                 flat_hbm.at[pl.ds(bi * OUT_DIM, OUT_DIM), :],
                                 wsems[p])
        return carry

    lax.fori_loop(0, BLK_PER_W // 2, pair, 0)

    for p in (0, 1):
        @pl.when(base + p < NBLK)
        def _():
            drain_wb(p)


def kernel(x, table):
    idx = x.T.reshape(-1).astype(jnp.int32)
    mesh = plsc.VectorSubcoreMesh(core_axis_name="c", subcore_axis_name="s")
    f_t = pl.kernel(
        _transpose_body,
        mesh=mesh,
        out_type=jax.ShapeDtypeStruct((250000, 128), jnp.float32),
        scratch_types=[
            pltpu.VMEM((OUT_DIM, CHUNK), jnp.float32),
            pltpu.VMEM((OUT_DIM, CHUNK), jnp.float32),
            pltpu.VMEM((OUT_DIM, CHUNK), jnp.float32),
            pltpu.VMEM((OUT_DIM, CHUNK), jnp.float32),
            pltpu.SemaphoreType.DMA,
            pltpu.SemaphoreType.DMA,
            pltpu.SemaphoreType.DMA,
            pltpu.SemaphoreType.DMA,
        ],
        compiler_params=pltpu.CompilerParams(
            use_tc_tiling_on_sc=True, needs_layout_passes=False),
    )
    tt = table.T
    tail = jnp.pad(tt[:, NBLK * CHUNK:], ((0, 0), (0, 64)))
    table2 = f_t(tt, tail)
    f = pl.kernel(
        _body,
        mesh=mesh,
        out_type=jax.ShapeDtypeStruct((N_FIELDS, OUT_DIM, BATCH), jnp.float32),
        scratch_types=[
            pltpu.VMEM((PER_W,), jnp.int32),
            pltpu.VMEM((PER_W,), jnp.int32),
            pltpu.VMEM((CHUNK, 128), jnp.float32),
            pltpu.VMEM((CHUNK, 128), jnp.float32),
            pltpu.VMEM((OUT_DIM, CHUNK), jnp.float32),
            pltpu.VMEM((OUT_DIM, CHUNK), jnp.float32),
            pltpu.SemaphoreType.DMA,
            pltpu.SemaphoreType.DMA,
            pltpu.SemaphoreType.DMA,
            pltpu.SemaphoreType.DMA,
            pltpu.SemaphoreType.DMA,
        ],
        compiler_params=pltpu.CompilerParams(
            use_tc_tiling_on_sc=True, needs_layout_passes=False),
    )
    out = f(table2, idx)
    return out.transpose(2, 0, 1)


# speedup vs baseline: 1.5719x; 1.0349x over previous
import jax
import jax.numpy as jnp
from jax import lax
from jax.experimental import pallas as pl
from jax.experimental.pallas import tpu as pltpu
from jax.experimental.pallas import tpu_sc as plsc

BATCH = 16384
N_FIELDS = 26
OUT_DIM = 32
NC = 2
NS = 16
NW = NC * NS
BPW = BATCH // NW
PER_W = BPW * N_FIELDS
CHUNK = 128
SUBC = BPW // CHUNK
L = 16


def _body(table_hbm, idx_hbm, out_hbm,
          idx_v, pid_v, gb0, gb1, ob0, ob1,
          gs0, gs1, ws0, ws1, isem):
    wid = lax.axis_index("s") * NC + lax.axis_index("c")
    b0 = wid * BPW

    for f in range(N_FIELDS):
        pltpu.async_copy(
            idx_hbm.at[pl.ds(f * BATCH + b0, BPW)],
            idx_v.at[pl.ds(f * BPW, BPW)], isem)
    pltpu.make_async_copy(idx_hbm.at[pl.ds(0, PER_W)], idx_v, isem).wait()

    def shift(i, c):
        pid_v[pl.ds(i * L, L)] = lax.shift_right_logical(idx_v[pl.ds(i * L, L)], 2)
        return c
    lax.fori_loop(0, PER_W // L, shift, 0)

    gbufs = (gb0, gb1)
    gsems = (gs0, gs1)
    obufs = (ob0, ob1)
    wsems = (ws0, ws1)

    def fire(t, p):
        pltpu.async_copy(
            table_hbm.at[pid_v.at[pl.ds(t * CHUNK, CHUNK)]],
            gbufs[p], gsems[p])

    def wait_gather(p):
        pltpu.make_async_copy(table_hbm.at[pl.ds(0, CHUNK)],
                              gbufs[p], gsems[p]).wait()

    lanes = lax.iota(jnp.int32, L)
    def extract(t, gp, op):
        gb = gbufs[gp]
        ob = obufs[op]

        @plsc.parallel_loop(0, CHUNK // L, unroll=2)
        def gbody(g):
            jv = idx_v[pl.ds(t * CHUNK + g * L, L)]
            qv = (jv & 3) * OUT_DIM
            rows = lanes + g * L
            for h in range(2):
                qh = qv + (h * L)
                for k in range(L):
                    rot = (lanes + k) & (L - 1)
                    vals = plsc.load_gather(gb, [rows, qh + rot])
                    plsc.store_scatter(ob, [rot + (h * L), rows], vals)

    def drain_wb(p):
        pltpu.make_async_copy(
            obufs[p], out_hbm.at[0, :, pl.ds(0, CHUNK)], wsems[p]).wait()

    fire(0, 0)
    fire(1, 1)
    n_chunks = N_FIELDS * SUBC

    def fpair(j, carry):
        for f_par in (0, 1):
            f = 2 * j + f_par
            for s in range(SUBC):
                t = f * SUBC + s
                gp = s % 2
                op = s % 2
                wait_gather(gp)
                @pl.when(t >= 2)
                def _():
                    drain_wb(op)
                extract(t, gp, op)
                @pl.when(t + 2 < n_chunks)
                def _():
                    fire(t + 2, s % 2)
                pltpu.async_copy(
                    obufs[op],
                    out_hbm.at[f, :, pl.ds(b0 + s * CHUNK, CHUNK)],
                    wsems[op])
        return carry

    lax.fori_loop(0, N_FIELDS // 2, fpair, 0)

    drain_wb(0)
    drain_wb(1)


NBLK = 7812
BLK_PER_W = 246


def _transpose_body(tt_hbm, tail_hbm, flat_hbm, ab0, ab1, tb0, tb1,
                    is0, is1, ws0, ws1):
    wid = lax.axis_index("s") * NC + lax.axis_index("c")
    base = wid * BLK_PER_W

    abufs = (ab0, ab1)
    tbufs = (tb0, tb1)
    isems = (is0, is1)
    wsems = (ws0, ws1)

    lanes = lax.iota(jnp.int32, L)
    l4 = lax.shift_right_logical(lanes, 2)
    la3 = (lanes & 3) * OUT_DIM
    cvecs = [lanes + c * L for c in range(8)]
    rvecs = [l4 + (c * 4) for c in range(8)]

    def fire_in(bi, p):
        pltpu.async_copy(tt_hbm.at[:, pl.ds(bi * CHUNK, CHUNK)],
                         abufs[p], isems[p])

    def wait_in(p):
        pltpu.make_async_copy(tt_hbm.at[:, pl.ds(0, CHUNK)],
                              abufs[p], isems[p]).wait()

    def drain_wb(p):
        pltpu.make_async_copy(tbufs[p], flat_hbm.at[pl.ds(0, OUT_DIM), :],
                              wsems[p]).wait()

    def shuffle(p):
        ab, tb = abufs[p], tbufs[p]

        @plsc.parallel_loop(0, L, unroll=2)
        def kstep(k):
            rot = (lanes + k) & (L - 1)
            for h in range(2):
                hrot = rot + h * L
                colv = la3 + (h * L) + rot
                for c in range(8):
                    vals = plsc.load_gather(ab, [hrot, cvecs[c]])
                    plsc.store_scatter(tb, [rvecs[c], colv], vals)

    @pl.when(wid == NW - 1)
    def _():
        pltpu.sync_copy(tail_hbm, ab0)

        @plsc.parallel_loop(0, L, unroll=2)
        def tstep(k):
            rot = (lanes + k) & (L - 1)
            for h in range(2):
                hrot = rot + h * L
                colv = la3 + (h * L) + rot
                for c in range(4):
                    vals = plsc.load_gather(ab0, [hrot, cvecs[c]])
                    plsc.store_scatter(tb0, [rvecs[c], colv], vals)
        pltpu.sync_copy(tb0.at[pl.ds(0, 16), :],
                        flat_hbm.at[pl.ds(NBLK * 32, 16), :])

    @pl.when(base < NBLK)
    def _():
        fire_in(base, 0)
    @pl.when(base + 1 < NBLK)
    def _():
        fire_in(base + 1, 1)

    def pair(i, carry):
        for p in (0, 1):
            j = 2 * i + p
            bi = base + j
            @pl.when(bi < NBLK)
            def _():
                wait_in(p)
                @pl.when(j >= 2)
                def _():
                    drain_wb(p)
                shuffle(p)
                @pl.when((j + 2 < BLK_PER_W) & (bi + 2 < NBLK))
                def _():
                    fire_in(bi + 2, p)
                pltpu.async_copy(tbufs[p],
                                 flat_hbm.at[pl.ds(bi * OUT_DIM, OUT_DIM), :],
                                 wsems[p])
        return carry

    lax.fori_loop(0, BLK_PER_W // 2, pair, 0)

    for p in (0, 1):
        @pl.when(base + p < NBLK)
        def _():
            drain_wb(p)


def kernel(x, table):
    idx = x.T.reshape(-1).astype(jnp.int32)
    mesh = plsc.VectorSubcoreMesh(core_axis_name="c", subcore_axis_name="s")
    f_t = pl.kernel(
        _transpose_body,
        mesh=mesh,
        out_type=jax.ShapeDtypeStruct((250000, 128), jnp.float32),
        scratch_types=[
            pltpu.VMEM((OUT_DIM, CHUNK), jnp.float32),
            pltpu.VMEM((OUT_DIM, CHUNK), jnp.float32),
            pltpu.VMEM((OUT_DIM, CHUNK), jnp.float32),
            pltpu.VMEM((OUT_DIM, CHUNK), jnp.float32),
            pltpu.SemaphoreType.DMA,
            pltpu.SemaphoreType.DMA,
            pltpu.SemaphoreType.DMA,
            pltpu.SemaphoreType.DMA,
        ],
        compiler_params=pltpu.CompilerParams(
            use_tc_tiling_on_sc=True, needs_layout_passes=False),
    )
    tt = table.T
    tail = jnp.pad(tt[:, NBLK * CHUNK:], ((0, 0), (0, 64)))
    table2 = f_t(tt, tail)
    f = pl.kernel(
        _body,
        mesh=mesh,
        out_type=jax.ShapeDtypeStruct((N_FIELDS, OUT_DIM, BATCH), jnp.float32),
        scratch_types=[
            pltpu.VMEM((PER_W,), jnp.int32),
            pltpu.VMEM((PER_W,), jnp.int32),
            pltpu.VMEM((CHUNK, 128), jnp.float32),
            pltpu.VMEM((CHUNK, 128), jnp.float32),
            pltpu.VMEM((OUT_DIM, CHUNK), jnp.float32),
            pltpu.VMEM((OUT_DIM, CHUNK), jnp.float32),
            pltpu.SemaphoreType.DMA,
            pltpu.SemaphoreType.DMA,
            pltpu.SemaphoreType.DMA,
            pltpu.SemaphoreType.DMA,
            pltpu.SemaphoreType.DMA,
        ],
        compiler_params=pltpu.CompilerParams(
            use_tc_tiling_on_sc=True, needs_layout_passes=False),
    )
    out = f(table2, idx)
    return out.transpose(2, 0, 1)
